# baseline (speedup 1.0000x reference)
import functools

import jax
import jax.numpy as jnp
from jax import lax
from jax.experimental import pallas as pl
from jax.experimental.pallas import tpu as pltpu
from jax.experimental.pallas import tpu_sc as plsc

BATCH = 16384
FEAT = 512
NC = 2
NS = 16
NW = NC * NS
ROWS_PER_W = BATCH // NW
CH = 32
NCHUNK = ROWS_PER_W // CH
NBUF_G = 4
NBUF_F = 2
LANES = 16
VECS_PER_ROW = FEAT // LANES


def _sc_body(feat_hbm, lab_hbm, cent_hbm, out_hbm,
             idx_v, rows_v, feat_v, out_v, *sems):
    wid = lax.axis_index("s") * NC + lax.axis_index("c")
    base = pl.multiple_of(wid * ROWS_PER_W, ROWS_PER_W)
    sems_g = sems[:NBUF_G]
    sems_f = sems[NBUF_G:]

    pltpu.sync_copy(lab_hbm.at[pl.ds(base, ROWS_PER_W)], idx_v)

    def start_g(c, b):
        r0 = pl.multiple_of(c * CH, CH)
        pltpu.async_copy(cent_hbm.at[idx_v.at[pl.ds(r0, CH)]],
                         rows_v.at[b], sems_g[b])

    def start_f(c, b):
        r0 = pl.multiple_of(c * CH, CH)
        pltpu.async_copy(feat_hbm.at[pl.ds(base + r0, CH)],
                         feat_v.at[b], sems_f[b])

    def wait(bg, bf):
        pltpu.make_async_copy(cent_hbm.at[pl.ds(0, CH)],
                              rows_v.at[bg], sems_g[bg]).wait()
        pltpu.make_async_copy(feat_hbm.at[pl.ds(0, CH)],
                              feat_v.at[bf], sems_f[bf]).wait()

    def compute(bg, bf, accs):
        def row_body(r, a):
            acc = list(a)
            for t in range(VECS_PER_ROW):
                f = feat_v[bf, r, pl.ds(t * LANES, LANES)]
                cv = rows_v[bg, r, pl.ds(t * LANES, LANES)]
                d = f - cv
                acc[t % 4] = acc[t % 4] + d * d
            return tuple(acc)
        return lax.fori_loop(0, CH, row_body, accs)

    for c in range(NBUF_G - 1):
        start_g(c, c)
    for c in range(NBUF_F - 1):
        start_f(c, c)
    zero = jnp.zeros((LANES,), jnp.float32)

    def outer(g, accs):
        c0 = g * NBUF_G
        for b in range(NBUF_G):
            c = c0 + b
            nxt_g = c + NBUF_G - 1
            nxt_f = c + NBUF_F - 1

            @pl.when(nxt_g < NCHUNK)
            def _():
                start_g(nxt_g, (b + NBUF_G - 1) % NBUF_G)

            @pl.when(nxt_f < NCHUNK)
            def _():
                start_f(nxt_f, (b + NBUF_F - 1) % NBUF_F)

            wait(b, b % NBUF_F)
            accs = compute(b, b % NBUF_F, accs)
        return accs

    a0, a1, a2, a3 = lax.fori_loop(0, NCHUNK // NBUF_G, outer,
                                   (zero, zero, zero, zero))
    out_v[...] = (a0 + a1) + (a2 + a3)
    pltpu.sync_copy(out_v, out_hbm.at[wid])


@jax.jit
def _center_loss_partials(features, labels, centers):
    mesh = plsc.VectorSubcoreMesh(core_axis_name="c", subcore_axis_name="s")
    run = pl.kernel(
        _sc_body,
        mesh=mesh,
        out_type=jax.ShapeDtypeStruct((NW, LANES), jnp.float32),
        scratch_types=[
            pltpu.VMEM((ROWS_PER_W,), jnp.int32),
            pltpu.VMEM((NBUF_G, CH, FEAT), jnp.float32),
            pltpu.VMEM((NBUF_F, CH, FEAT), jnp.float32),
            pltpu.VMEM((LANES,), jnp.float32),
        ] + [pltpu.SemaphoreType.DMA] * (NBUF_G + NBUF_F),
    )
    return run(features, labels, centers)


def kernel(features, labels, centers):
    partials = _center_loss_partials(
        features, labels.astype(jnp.int32), centers)
    return jnp.sum(partials) / jnp.float32(BATCH * FEAT)


# speedup vs baseline: 1.0469x; 1.0469x over previous
import functools

import jax
import jax.numpy as jnp
from jax import lax
from jax.experimental import pallas as pl
from jax.experimental.pallas import tpu as pltpu
from jax.experimental.pallas import tpu_sc as plsc

BATCH = 16384
FEAT = 512
NC = 2
NS = 16
NW = NC * NS
ROWS_PER_W = BATCH // NW
CH = 16
NCHUNK = ROWS_PER_W // CH
NBUF_G = 8
NBUF_F = 4
LANES = 16
VECS_PER_ROW = FEAT // LANES


def _sc_body(feat_hbm, lab_hbm, cent_hbm, out_hbm,
             idx_v, rows_v, feat_v, out_v, *sems):
    wid = lax.axis_index("s") * NC + lax.axis_index("c")
    base = pl.multiple_of(wid * ROWS_PER_W, ROWS_PER_W)
    sems_g = sems[:NBUF_G]
    sems_f = sems[NBUF_G:]

    pltpu.sync_copy(lab_hbm.at[pl.ds(base, ROWS_PER_W)], idx_v)

    def start_g(c, b):
        r0 = pl.multiple_of(c * CH, CH)
        pltpu.async_copy(cent_hbm.at[idx_v.at[pl.ds(r0, CH)]],
                         rows_v.at[b], sems_g[b])

    def start_f(c, b):
        r0 = pl.multiple_of(c * CH, CH)
        pltpu.async_copy(feat_hbm.at[pl.ds(base + r0, CH)],
                         feat_v.at[b], sems_f[b])

    def wait(bg, bf):
        pltpu.make_async_copy(cent_hbm.at[pl.ds(0, CH)],
                              rows_v.at[bg], sems_g[bg]).wait()
        pltpu.make_async_copy(feat_hbm.at[pl.ds(0, CH)],
                              feat_v.at[bf], sems_f[bf]).wait()

    def compute(bg, bf, accs):
        def row_body(r, a):
            acc = list(a)
            for t in range(VECS_PER_ROW):
                f = feat_v[bf, r, pl.ds(t * LANES, LANES)]
                cv = rows_v[bg, r, pl.ds(t * LANES, LANES)]
                d = f - cv
                acc[t % 4] = acc[t % 4] + d * d
            return tuple(acc)
        return lax.fori_loop(0, CH, row_body, accs)

    for c in range(NBUF_G - 1):
        start_g(c, c)
    for c in range(NBUF_F - 1):
        start_f(c, c)
    zero = jnp.zeros((LANES,), jnp.float32)

    def outer(g, accs):
        c0 = g * NBUF_G
        for b in range(NBUF_G):
            c = c0 + b
            nxt_g = c + NBUF_G - 1
            nxt_f = c + NBUF_F - 1

            @pl.when(nxt_g < NCHUNK)
            def _():
                start_g(nxt_g, (b + NBUF_G - 1) % NBUF_G)

            @pl.when(nxt_f < NCHUNK)
            def _():
                start_f(nxt_f, (b + NBUF_F - 1) % NBUF_F)

            wait(b, b % NBUF_F)
            accs = compute(b, b % NBUF_F, accs)
        return accs

    a0, a1, a2, a3 = lax.fori_loop(0, NCHUNK // NBUF_G, outer,
                                   (zero, zero, zero, zero))
    out_v[...] = (a0 + a1) + (a2 + a3)
    pltpu.sync_copy(out_v, out_hbm.at[wid])


@jax.jit
def _center_loss_partials(features, labels, centers):
    mesh = plsc.VectorSubcoreMesh(core_axis_name="c", subcore_axis_name="s")
    run = pl.kernel(
        _sc_body,
        mesh=mesh,
        out_type=jax.ShapeDtypeStruct((NW, LANES), jnp.float32),
        scratch_types=[
            pltpu.VMEM((ROWS_PER_W,), jnp.int32),
            pltpu.VMEM((NBUF_G, CH, FEAT), jnp.float32),
            pltpu.VMEM((NBUF_F, CH, FEAT), jnp.float32),
            pltpu.VMEM((LANES,), jnp.float32),
        ] + [pltpu.SemaphoreType.DMA] * (NBUF_G + NBUF_F),
    )
    return run(features, labels, centers)


def kernel(features, labels, centers):
    partials = _center_loss_partials(
        features, labels.astype(jnp.int32), centers)
    return jnp.sum(partials) / jnp.float32(BATCH * FEAT)


# speedup vs baseline: 1.1159x; 1.0659x over previous
import functools

import jax
import jax.numpy as jnp
from jax import lax
from jax.experimental import pallas as pl
from jax.experimental.pallas import tpu as pltpu
from jax.experimental.pallas import tpu_sc as plsc

BATCH = 16384
FEAT = 512
NC = 2
NS = 16
NW = NC * NS
ROWS_PER_W = BATCH // NW
CH = 16
NCHUNK = ROWS_PER_W // CH
NBUF_G = 4
NBUF_F = 4
LANES = 16
VECS_PER_ROW = FEAT // LANES


def _sc_body(feat_hbm, lab_hbm, cent_hbm, out_hbm,
             idx_v, rows_v, feat_v, out_v, *sems):
    wid = lax.axis_index("s") * NC + lax.axis_index("c")
    base = pl.multiple_of(wid * ROWS_PER_W, ROWS_PER_W)
    sems_g = sems[:NBUF_G]
    sems_f = sems[NBUF_G:]

    pltpu.sync_copy(lab_hbm.at[pl.ds(base, ROWS_PER_W)], idx_v)

    def start_g(c, b):
        r0 = pl.multiple_of(c * CH, CH)
        pltpu.async_copy(cent_hbm.at[idx_v.at[pl.ds(r0, CH)]],
                         rows_v.at[b], sems_g[b])

    def start_f(c, b):
        r0 = pl.multiple_of(c * CH, CH)
        pltpu.async_copy(feat_hbm.at[pl.ds(base + r0, CH)],
                         feat_v.at[b], sems_f[b])

    def wait(bg, bf):
        pltpu.make_async_copy(cent_hbm.at[pl.ds(0, CH)],
                              rows_v.at[bg], sems_g[bg]).wait()
        pltpu.make_async_copy(feat_hbm.at[pl.ds(0, CH)],
                              feat_v.at[bf], sems_f[bf]).wait()

    def compute(bg, bf, accs):
        def row_body(r, a):
            acc = list(a)
            for t in range(VECS_PER_ROW):
                f = feat_v[bf, r, pl.ds(t * LANES, LANES)]
                cv = rows_v[bg, r, pl.ds(t * LANES, LANES)]
                d = f - cv
                acc[t % 4] = acc[t % 4] + d * d
            return tuple(acc)
        return lax.fori_loop(0, CH, row_body, accs)

    for c in range(NBUF_G - 1):
        start_g(c, c)
    for c in range(NBUF_F - 1):
        start_f(c, c)
    zero = jnp.zeros((LANES,), jnp.float32)

    def outer(g, accs):
        c0 = g * NBUF_G
        for b in range(NBUF_G):
            c = c0 + b
            nxt_g = c + NBUF_G - 1
            nxt_f = c + NBUF_F - 1

            @pl.when(nxt_g < NCHUNK)
            def _():
                start_g(nxt_g, (b + NBUF_G - 1) % NBUF_G)

            @pl.when(nxt_f < NCHUNK)
            def _():
                start_f(nxt_f, (b + NBUF_F - 1) % NBUF_F)

            wait(b, b % NBUF_F)
            accs = compute(b, b % NBUF_F, accs)
        return accs

    a0, a1, a2, a3 = lax.fori_loop(0, NCHUNK // NBUF_G, outer,
                                   (zero, zero, zero, zero))
    out_v[...] = (a0 + a1) + (a2 + a3)
    pltpu.sync_copy(out_v, out_hbm.at[wid])


@jax.jit
def _center_loss_partials(features, labels, centers):
    mesh = plsc.VectorSubcoreMesh(core_axis_name="c", subcore_axis_name="s")
    run = pl.kernel(
        _sc_body,
        mesh=mesh,
        out_type=jax.ShapeDtypeStruct((NW, LANES), jnp.float32),
        scratch_types=[
            pltpu.VMEM((ROWS_PER_W,), jnp.int32),
            pltpu.VMEM((NBUF_G, CH, FEAT), jnp.float32),
            pltpu.VMEM((NBUF_F, CH, FEAT), jnp.float32),
            pltpu.VMEM((LANES,), jnp.float32),
        ] + [pltpu.SemaphoreType.DMA] * (NBUF_G + NBUF_F),
    )
    return run(features, labels, centers)


def kernel(features, labels, centers):
    partials = _center_loss_partials(
        features, labels.astype(jnp.int32), centers)
    return jnp.sum(partials) / jnp.float32(BATCH * FEAT)


# speedup vs baseline: 1.1208x; 1.0044x over previous
import functools

import jax
import jax.numpy as jnp
from jax import lax
from jax.experimental import pallas as pl
from jax.experimental.pallas import tpu as pltpu
from jax.experimental.pallas import tpu_sc as plsc

BATCH = 16384
FEAT = 512
NC = 2
NS = 16
NW = NC * NS
ROWS_PER_W = BATCH // NW
CH = 16
NCHUNK = ROWS_PER_W // CH
NBUF_G = 4
NBUF_F = 4
LANES = 16
VECS_PER_ROW = FEAT // LANES


def _sc_body(feat_hbm, lab_hbm, cent_hbm, out_hbm,
             idx_v, rows_v, feat_v, out_v, *sems):
    wid = lax.axis_index("s") * NC + lax.axis_index("c")
    base = pl.multiple_of(wid * ROWS_PER_W, ROWS_PER_W)
    sems_g = sems[:NBUF_G]
    sems_f = sems[NBUF_G:]

    pltpu.sync_copy(lab_hbm.at[pl.ds(base, ROWS_PER_W)], idx_v)

    def start_g(c, b):
        r0 = pl.multiple_of(c * CH, CH)
        pltpu.async_copy(cent_hbm.at[idx_v.at[pl.ds(r0, CH)]],
                         rows_v.at[b], sems_g[b])

    def start_f(c, b):
        r0 = pl.multiple_of(c * CH, CH)
        pltpu.async_copy(feat_hbm.at[pl.ds(base + r0, CH)],
                         feat_v.at[b], sems_f[b])

    def wait(bg, bf):
        pltpu.make_async_copy(cent_hbm.at[pl.ds(0, CH)],
                              rows_v.at[bg], sems_g[bg]).wait()
        pltpu.make_async_copy(feat_hbm.at[pl.ds(0, CH)],
                              feat_v.at[bf], sems_f[bf]).wait()

    def compute(bg, bf, accs):
        @plsc.parallel_loop(0, CH, carry=accs)
        def row_body(r, a):
            acc = list(a)
            for t in range(VECS_PER_ROW):
                f = feat_v[bf, r, pl.ds(t * LANES, LANES)]
                cv = rows_v[bg, r, pl.ds(t * LANES, LANES)]
                d = f - cv
                acc[t % 4] = acc[t % 4] + d * d
            return tuple(acc)
        return row_body

    for c in range(NBUF_G - 1):
        start_g(c, c)
    for c in range(NBUF_F - 1):
        start_f(c, c)
    zero = jnp.zeros((LANES,), jnp.float32)

    def outer(g, accs):
        c0 = g * NBUF_G
        for b in range(NBUF_G):
            c = c0 + b
            nxt_g = c + NBUF_G - 1
            nxt_f = c + NBUF_F - 1

            @pl.when(nxt_g < NCHUNK)
            def _():
                start_g(nxt_g, (b + NBUF_G - 1) % NBUF_G)

            @pl.when(nxt_f < NCHUNK)
            def _():
                start_f(nxt_f, (b + NBUF_F - 1) % NBUF_F)

            wait(b, b % NBUF_F)
            accs = compute(b, b % NBUF_F, accs)
        return accs

    a0, a1, a2, a3 = lax.fori_loop(0, NCHUNK // NBUF_G, outer,
                                   (zero, zero, zero, zero))
    out_v[...] = (a0 + a1) + (a2 + a3)
    pltpu.sync_copy(out_v, out_hbm.at[wid])


@jax.jit
def _center_loss_partials(features, labels, centers):
    mesh = plsc.VectorSubcoreMesh(core_axis_name="c", subcore_axis_name="s")
    run = pl.kernel(
        _sc_body,
        mesh=mesh,
        out_type=jax.ShapeDtypeStruct((NW, LANES), jnp.float32),
        scratch_types=[
            pltpu.VMEM((ROWS_PER_W,), jnp.int32),
            pltpu.VMEM((NBUF_G, CH, FEAT), jnp.float32),
            pltpu.VMEM((NBUF_F, CH, FEAT), jnp.float32),
            pltpu.VMEM((LANES,), jnp.float32),
        ] + [pltpu.SemaphoreType.DMA] * (NBUF_G + NBUF_F),
    )
    return run(features, labels, centers)


def kernel(features, labels, centers):
    partials = _center_loss_partials(
        features, labels.astype(jnp.int32), centers)
    return jnp.sum(partials) / jnp.float32(BATCH * FEAT)
